# Initial kernel scaffold; baseline (speedup 1.0000x reference)
import jax
import jax.numpy as jnp
from jax.experimental import pallas as pl


def kernel(x, adj):
    raise NotImplementedError("write your pallas kernel here")


# speedup vs baseline: 1.0992x; 1.0992x over previous
import jax
import jax.numpy as jnp
from jax.experimental import pallas as pl
from jax.experimental.pallas import tpu as pltpu

_MIN_NORM = 1e-15
_EPS = 4e-3


def _artanh(v):
    v = jnp.clip(v, -1.0 + 1e-7, 1.0 - 1e-7)
    return 0.5 * (jnp.log1p(v) - jnp.log1p(-v))


def _tangent_body(x_ref, o_ref):
    x = x_ref[...]
    n = jnp.sqrt(jnp.sum(x * x, axis=-1, keepdims=True))
    n = jnp.maximum(n, _MIN_NORM)
    o_ref[...] = x / n * _artanh(n)


def _agg_body(xt_ref, adj_ref, o_ref):
    acc = jnp.dot(adj_ref[...], xt_ref[...],
                  preferred_element_type=jnp.float32)
    n = jnp.maximum(jnp.sqrt(jnp.sum(acc * acc, axis=-1, keepdims=True)),
                    _MIN_NORM)
    y = jnp.tanh(n) * acc / n
    yn = jnp.maximum(jnp.sqrt(jnp.sum(y * y, axis=-1, keepdims=True)),
                     _MIN_NORM)
    maxnorm = 1.0 - _EPS
    o_ref[...] = jnp.where(yn > maxnorm, y / yn * maxnorm, y)


def kernel(x, adj):
    n_nodes, d = x.shape
    bt = n_nodes // 10 if n_nodes % 10 == 0 else n_nodes
    xt = pl.pallas_call(
        _tangent_body,
        grid=(n_nodes // bt,),
        in_specs=[pl.BlockSpec((bt, d), lambda i: (i, 0))],
        out_specs=pl.BlockSpec((bt, d), lambda i: (i, 0)),
        out_shape=jax.ShapeDtypeStruct((n_nodes, d), jnp.float32),
    )(x)

    bm = 400 if n_nodes % 400 == 0 else n_nodes
    out = pl.pallas_call(
        _agg_body,
        grid=(n_nodes // bm,),
        in_specs=[
            pl.BlockSpec((n_nodes, d), lambda i: (0, 0)),
            pl.BlockSpec((bm, n_nodes), lambda i: (i, 0)),
        ],
        out_specs=pl.BlockSpec((bm, d), lambda i: (i, 0)),
        out_shape=jax.ShapeDtypeStruct((n_nodes, d), jnp.float32),
    )(xt, adj)
    return out
